# baseline (speedup 1.0000x reference)
import jax
import jax.numpy as jnp
from jax import lax
from jax.experimental import pallas as pl
from jax.experimental.pallas import tpu as pltpu
from jax.experimental.pallas import tpu_sc as plsc

N = 10000
E = 320000
D = 128
DH = D // 2

NC = 2
NS = 16
K = 128
NPAIR = 80
EPW = NPAIR * 2 * K
EPAD = NS * EPW
NPAD = 10240
RPT = NPAD // NS


def _matmul_body(x_ref, w_ref, o_ref):
    s = jnp.dot(x_ref[...], w_ref[...], preferred_element_type=jnp.float32)
    o_ref[0] = s[:, :DH]
    o_ref[1] = s[:, DH:]


def _matmul(x, W):
    bm = 400
    return pl.pallas_call(
        _matmul_body,
        grid=(N // bm,),
        in_specs=[
            pl.BlockSpec((bm, D), lambda i: (i, 0)),
            pl.BlockSpec((D, D), lambda i: (0, 0)),
        ],
        out_specs=pl.BlockSpec((NC, bm, DH), lambda i: (0, i, 0)),
        out_shape=jax.ShapeDtypeStruct((NC, N, DH), jnp.float32),
    )(x, W)


def _spmm_kernel(support, row, col, w, bias, out, acc,
                 colA, rowA, wA, colB, rowB, wB,
                 rows0, rows1, bv, gsem0, gsem1, isemA, isemB, bsem):
    cid = lax.axis_index("c")
    sid = lax.axis_index("s")
    cbase = jnp.full((16,), cid * N, jnp.int32)

    pltpu.async_copy(bias.at[pl.ds(cid, 1)], bv, bsem)
    pltpu.make_async_copy(bias, bv, bsem).wait()
    bregs = [bv[0, pl.ds(16 * t, 16)] for t in range(DH // 16)]

    def brow(r, _):
        for t in range(DH // 16):
            rows0[r, pl.ds(16 * t, 16)] = bregs[t]
        return 0

    lax.fori_loop(0, K, brow, 0)
    for z in range(RPT // K):
        pltpu.sync_copy(rows0, acc.at[pl.ds(sid * RPT + z * K, K)])

    def load_set(pp, c, r, ww, sem):
        pltpu.async_copy(col.at[sid, pp], c, sem)
        pltpu.async_copy(row.at[sid, pp], r, sem)
        pltpu.async_copy(w.at[sid, pp], ww, sem)

    def wait_set(c, r, ww, sem):
        pltpu.make_async_copy(col, c, sem).wait()
        pltpu.make_async_copy(row, r, sem).wait()
        pltpu.make_async_copy(w, ww, sem).wait()
        for j in range(2):
            for t in range(K // 16):
                c[j, pl.ds(16 * t, 16)] = c[j, pl.ds(16 * t, 16)] + cbase

    def gather(cset, j, buf, gsem):
        pltpu.async_copy(support.at[cset.at[j]], buf, gsem)

    def gwait(buf, gsem):
        pltpu.make_async_copy(support, buf, gsem).wait()

    def scale_scatter(rset, wset, j, buf):
        def scale(g, _):
            wg = wset[j, pl.ds(16 * g, 16)]
            for e16 in range(16):
                wb = jnp.broadcast_to(wg[e16], (16,))
                e = 16 * g + e16
                for jj in range(DH // 16):
                    buf[e, pl.ds(16 * jj, 16)] = (
                        buf[e, pl.ds(16 * jj, 16)] * wb)
            return 0

        lax.fori_loop(0, K // 16, scale, 0)
        pltpu.sync_copy(buf, acc.at[rset.at[j]], add=True)

    load_set(0, colA, rowA, wA, isemA)
    load_set(1, colB, rowB, wB, isemB)
    plsc.subcore_barrier()
    wait_set(colA, rowA, wA, isemA)
    gather(colA, 0, rows0, gsem0)

    def one_pair(pp, cX, rX, wX, isemX, cY, rY, wY, isemY):
        gather(cX, 1, rows1, gsem1)
        gwait(rows0, gsem0)
        scale_scatter(rX, wX, 0, rows0)
        gwait(rows1, gsem1)
        scale_scatter(rX, wX, 1, rows1)
        wait_set(cY, rY, wY, isemY)
        gather(cY, 0, rows0, gsem0)
        load_set(lax.rem(pp + 2, NPAIR), cX, rX, wX, isemX)

    def super_body(q, _):
        one_pair(2 * q, colA, rowA, wA, isemA, colB, rowB, wB, isemB)
        one_pair(2 * q + 1, colB, rowB, wB, isemB, colA, rowA, wA, isemA)
        return 0

    lax.fori_loop(0, NPAIR // 2, super_body, 0)
    gwait(rows0, gsem0)
    wait_set(colB, rowB, wB, isemB)

    plsc.subcore_barrier()

    pltpu.sync_copy(acc.at[pl.ds(sid * RPT, RPT)],
                    out.at[cid, pl.ds(sid * RPT, RPT)])


def _spmm_sc(support2, row4, col4, w4, bias2):
    mesh = plsc.VectorSubcoreMesh(core_axis_name="c", subcore_axis_name="s")
    return pl.kernel(
        _spmm_kernel,
        out_type=jax.ShapeDtypeStruct((NC, NPAD, DH), jnp.float32),
        mesh=mesh,
        compiler_params=pltpu.CompilerParams(use_tc_tiling_on_sc=False),
        scratch_types=[
            pltpu.VMEM_SHARED((NPAD, DH), jnp.float32),
            pltpu.VMEM((2, K), jnp.int32),
            pltpu.VMEM((2, K), jnp.int32),
            pltpu.VMEM((2, K), jnp.float32),
            pltpu.VMEM((2, K), jnp.int32),
            pltpu.VMEM((2, K), jnp.int32),
            pltpu.VMEM((2, K), jnp.float32),
            pltpu.VMEM((K, DH), jnp.float32),
            pltpu.VMEM((K, DH), jnp.float32),
            pltpu.VMEM((1, DH), jnp.float32),
            pltpu.SemaphoreType.DMA,
            pltpu.SemaphoreType.DMA,
            pltpu.SemaphoreType.DMA,
            pltpu.SemaphoreType.DMA,
            pltpu.SemaphoreType.DMA,
        ],
    )(support2, row4, col4, w4, bias2)


@jax.jit
def kernel(x, edge_index, edge_weight, W, b):
    support2 = _matmul(x, W).reshape(NC * N, DH)
    npadE = EPAD - E
    fill = jnp.arange(npadE, dtype=jnp.int32)
    row_p = jnp.concatenate([edge_index[0], N + fill % (NPAD - N)])
    col_p = jnp.concatenate([edge_index[1], fill % N])
    w_p = jnp.concatenate([edge_weight, jnp.zeros((npadE,), jnp.float32)])
    row4 = row_p.reshape(NS, NPAIR, 2, K)
    col4 = col_p.reshape(NS, NPAIR, 2, K)
    w4 = w_p.reshape(NS, NPAIR, 2, K)
    bias2 = b.reshape(NC, DH)
    out3 = _spmm_sc(support2, row4, col4, w4, bias2)
    return jnp.concatenate([out3[0, :N], out3[1, :N]], axis=1)


# speedup vs baseline: 2.1252x; 2.1252x over previous
import jax
import jax.numpy as jnp
from jax import lax
from jax.experimental import pallas as pl
from jax.experimental.pallas import tpu as pltpu
from jax.experimental.pallas import tpu_sc as plsc

N = 10000
E = 320000
D = 128

NC = 2
NS = 16
NW = NC * NS
K = 80
NQUAD = 32
EPW = NQUAD * 4 * K
EPAD = NW * EPW
NPAD = 10240
RPT = NPAD // NS


def _matmul_body(x_ref, w_ref, o_ref):
    o_ref[...] = jnp.dot(x_ref[...], w_ref[...],
                         preferred_element_type=jnp.float32)


def _matmul(x, W):
    bm = 400
    return pl.pallas_call(
        _matmul_body,
        grid=(N // bm,),
        in_specs=[
            pl.BlockSpec((bm, D), lambda i: (i, 0)),
            pl.BlockSpec((D, D), lambda i: (0, 0)),
        ],
        out_specs=pl.BlockSpec((bm, D), lambda i: (i, 0)),
        out_shape=jax.ShapeDtypeStruct((N, D), jnp.float32),
    )(x, W)


def _spmm_kernel(support, row, col, w, out, acc,
                 colA, rowA, wA, colB, rowB, wB,
                 b0, b1, b2, b3, pidx,
                 g0, g1, g2, g3, s0, s1, s2, s3, isemA, isemB):
    cid = lax.axis_index("c")
    sid = lax.axis_index("s")
    wid = sid * NC + cid

    bufs = (b0, b1, b2, b3)
    gsems = (g0, g1, g2, g3)
    ssems = (s0, s1, s2, s3)

    zeros16 = jnp.zeros((16,), jnp.float32)

    def zrow_b(buf):
        def zrow(r, _):
            for j in range(D // 16):
                buf[r, pl.ds(16 * j, 16)] = zeros16
            return 0
        lax.fori_loop(0, K, zrow, 0)

    zrow_b(b0)
    for z in range(RPT // K):
        pltpu.sync_copy(b0, acc.at[pl.ds(sid * RPT + z * K, K)])

    pbase = jnp.broadcast_to(N + sid * 10, (16,)).astype(jnp.int32)
    for t in range(K // 16):
        pidx[0, pl.ds(16 * t, 16)] = (
            lax.iota(jnp.int32, 16) + pbase + 16 * t)

    def load_set(qq, c, r, ww, sem):
        pltpu.async_copy(col.at[wid, qq], c, sem)
        pltpu.async_copy(row.at[wid, qq], r, sem)
        pltpu.async_copy(w.at[wid, qq], ww, sem)

    def wait_set(c, r, ww, sem):
        pltpu.make_async_copy(col, c, sem).wait()
        pltpu.make_async_copy(row, r, sem).wait()
        pltpu.make_async_copy(w, ww, sem).wait()

    def gather(cset, j, buf, gsem):
        pltpu.async_copy(support.at[cset.at[j]], buf, gsem)

    def gwait(buf, gsem):
        pltpu.make_async_copy(support, buf, gsem).wait()

    def scatter(rset, j, buf, ssem):
        pltpu.async_copy(buf, acc.at[rset.at[j]], ssem, add=True)

    def swait(buf, ssem):
        pltpu.make_async_copy(support, buf, ssem).wait()

    def scale(wset, j, buf):
        def body(g, _):
            wg = wset[j, pl.ds(16 * g, 16)]
            for e16 in range(16):
                wb = jnp.broadcast_to(wg[e16], (16,))
                e = 16 * g + e16
                for jj in range(D // 16):
                    buf[e, pl.ds(16 * jj, 16)] = (
                        buf[e, pl.ds(16 * jj, 16)] * wb)
            return 0
        lax.fori_loop(0, K // 16, body, 0)

    load_set(0, colA, rowA, wA, isemA)
    load_set(1, colB, rowB, wB, isemB)
    zrow_b(b2)
    zrow_b(b3)
    plsc.subcore_barrier()
    scatter(pidx, 0, b2, s2)
    scatter(pidx, 0, b3, s3)
    wait_set(colA, rowA, wA, isemA)
    gather(colA, 0, b0, g0)
    gather(colA, 1, b1, g1)

    def quad(qq, cX, rX, wX, isemX, cY, rY, wY, isemY):
        gwait(b0, g0)
        scale(wX, 0, b0)
        swait(b2, s2)
        gather(cX, 2, b2, g2)
        scatter(rX, 0, b0, s0)
        gwait(b1, g1)
        scale(wX, 1, b1)
        swait(b3, s3)
        gather(cX, 3, b3, g3)
        scatter(rX, 1, b1, s1)
        gwait(b2, g2)
        scale(wX, 2, b2)
        scatter(rX, 2, b2, s2)
        gwait(b3, g3)
        scale(wX, 3, b3)
        scatter(rX, 3, b3, s3)
        wait_set(cY, rY, wY, isemY)
        swait(b0, s0)
        gather(cY, 0, b0, g0)
        swait(b1, s1)
        gather(cY, 1, b1, g1)
        load_set(lax.rem(qq + 2, NQUAD), cX, rX, wX, isemX)

    def super_body(p, _):
        quad(2 * p, colA, rowA, wA, isemA, colB, rowB, wB, isemB)
        quad(2 * p + 1, colB, rowB, wB, isemB, colA, rowA, wA, isemA)
        return 0

    lax.fori_loop(0, NQUAD // 2, super_body, 0)
    gwait(b0, g0)
    gwait(b1, g1)
    swait(b2, s2)
    swait(b3, s3)
    wait_set(colB, rowB, wB, isemB)

    plsc.subcore_barrier()

    base = cid * NPAD + sid * RPT
    pltpu.sync_copy(acc.at[pl.ds(sid * RPT, RPT)], out.at[pl.ds(base, RPT)])


def _spmm_sc(support, row4, col4, w4):
    mesh = plsc.VectorSubcoreMesh(core_axis_name="c", subcore_axis_name="s")
    return pl.kernel(
        _spmm_kernel,
        out_type=jax.ShapeDtypeStruct((NC * NPAD, D), jnp.float32),
        mesh=mesh,
        scratch_types=[
            pltpu.VMEM_SHARED((NPAD, D), jnp.float32),
            pltpu.VMEM((4, K), jnp.int32),
            pltpu.VMEM((4, K), jnp.int32),
            pltpu.VMEM((4, K), jnp.float32),
            pltpu.VMEM((4, K), jnp.int32),
            pltpu.VMEM((4, K), jnp.int32),
            pltpu.VMEM((4, K), jnp.float32),
            pltpu.VMEM((K, D), jnp.float32),
            pltpu.VMEM((K, D), jnp.float32),
            pltpu.VMEM((K, D), jnp.float32),
            pltpu.VMEM((K, D), jnp.float32),
            pltpu.VMEM((1, K), jnp.int32),
            pltpu.SemaphoreType.DMA,
            pltpu.SemaphoreType.DMA,
            pltpu.SemaphoreType.DMA,
            pltpu.SemaphoreType.DMA,
            pltpu.SemaphoreType.DMA,
            pltpu.SemaphoreType.DMA,
            pltpu.SemaphoreType.DMA,
            pltpu.SemaphoreType.DMA,
            pltpu.SemaphoreType.DMA,
            pltpu.SemaphoreType.DMA,
        ],
    )(support, row4, col4, w4)


def _combine_body(p0_ref, p1_ref, b_ref, o_ref):
    o_ref[...] = p0_ref[...] + p1_ref[...] + b_ref[...]


def _combine(partials, b2d):
    bm = 80
    return pl.pallas_call(
        _combine_body,
        grid=(N // bm,),
        in_specs=[
            pl.BlockSpec((bm, D), lambda i: (i, 0)),
            pl.BlockSpec((bm, D), lambda i: (i + NPAD // bm, 0)),
            pl.BlockSpec((1, D), lambda i: (0, 0)),
        ],
        out_specs=pl.BlockSpec((bm, D), lambda i: (i, 0)),
        out_shape=jax.ShapeDtypeStruct((N, D), jnp.float32),
    )(partials, partials, b2d)


@jax.jit
def kernel(x, edge_index, edge_weight, W, b):
    support = _matmul(x, W)
    npadE = EPAD - E
    fill = jnp.arange(npadE, dtype=jnp.int32)
    row_p = jnp.concatenate([edge_index[0], N + fill % (NPAD - N)])
    col_p = jnp.concatenate([edge_index[1], fill % N])
    w_p = jnp.concatenate([edge_weight, jnp.zeros((npadE,), jnp.float32)])
    row4 = row_p.reshape(NW, NQUAD, 4, K)
    col4 = col_p.reshape(NW, NQUAD, 4, K)
    w4 = w_p.reshape(NW, NQUAD, 4, K)
    partials = _spmm_sc(support, row4, col4, w4)
    return _combine(partials, b.reshape(1, D))


# speedup vs baseline: 2.8285x; 1.3310x over previous
import jax
import jax.numpy as jnp
from jax import lax
from jax.experimental import pallas as pl
from jax.experimental.pallas import tpu as pltpu
from jax.experimental.pallas import tpu_sc as plsc

N = 10000
E = 320000
D = 128

NC = 2
NS = 16
NW = NC * NS
K = 80
NQUAD = 32
EPW = NQUAD * 4 * K
EPAD = NW * EPW
NPAD = 10240
RPT = NPAD // NS


def _matmul_body(x_ref, w_ref, o_ref):
    o_ref[...] = jnp.dot(x_ref[...], w_ref[...],
                         preferred_element_type=jnp.float32)


def _matmul(x, W):
    bm = 2000
    return pl.pallas_call(
        _matmul_body,
        grid=(N // bm,),
        in_specs=[
            pl.BlockSpec((bm, D), lambda i: (i, 0)),
            pl.BlockSpec((D, D), lambda i: (0, 0)),
        ],
        out_specs=pl.BlockSpec((bm, D), lambda i: (i, 0)),
        out_shape=jax.ShapeDtypeStruct((N, D), jnp.float32),
    )(x, W)


def _spmm_kernel(support, row, col, w, out0, out1, acc,
                 colA, rowA, wA, colB, rowB, wB,
                 b0, b1, b2, b3, pidx,
                 g0, g1, g2, g3, s0, s1, s2, s3, isemA, isemB):
    cid = lax.axis_index("c")
    sid = lax.axis_index("s")
    wid = sid * NC + cid

    bufs = (b0, b1, b2, b3)
    gsems = (g0, g1, g2, g3)
    ssems = (s0, s1, s2, s3)

    zeros16 = jnp.zeros((16,), jnp.float32)

    def zrow_b(buf):
        def zrow(r, _):
            for j in range(D // 16):
                buf[r, pl.ds(16 * j, 16)] = zeros16
            return 0
        lax.fori_loop(0, K, zrow, 0)

    zrow_b(b0)
    for z in range(RPT // K):
        pltpu.sync_copy(b0, acc.at[pl.ds(sid * RPT + z * K, K)])

    pbase = jnp.broadcast_to(N + sid * 10, (16,)).astype(jnp.int32)
    for t in range(K // 16):
        pidx[0, pl.ds(16 * t, 16)] = (
            lax.iota(jnp.int32, 16) + pbase + 16 * t)

    def load_set(qq, c, r, ww, sem):
        pltpu.async_copy(col.at[wid, qq], c, sem)
        pltpu.async_copy(row.at[wid, qq], r, sem)
        pltpu.async_copy(w.at[wid, qq], ww, sem)

    def wait_set(c, r, ww, sem):
        pltpu.make_async_copy(col, c, sem).wait()
        pltpu.make_async_copy(row, r, sem).wait()
        pltpu.make_async_copy(w, ww, sem).wait()

    def gather(cset, j, buf, gsem):
        pltpu.async_copy(support.at[cset.at[j]], buf, gsem)

    def gwait(buf, gsem):
        pltpu.make_async_copy(support, buf, gsem).wait()

    def scatter(rset, j, buf, ssem):
        pltpu.async_copy(buf, acc.at[rset.at[j]], ssem, add=True)

    def swait(buf, ssem):
        pltpu.make_async_copy(support, buf, ssem).wait()

    def scale(wset, j, buf):
        def body(g, _):
            wg = wset[j, pl.ds(16 * g, 16)]
            for e16 in range(16):
                wb = jnp.broadcast_to(wg[e16], (16,))
                e = 16 * g + e16
                for jj in range(D // 16):
                    buf[e, pl.ds(16 * jj, 16)] = (
                        buf[e, pl.ds(16 * jj, 16)] * wb)
            return 0
        lax.fori_loop(0, K // 16, body, 0)

    load_set(0, colA, rowA, wA, isemA)
    load_set(1, colB, rowB, wB, isemB)
    zrow_b(b2)
    zrow_b(b3)
    plsc.subcore_barrier()
    scatter(pidx, 0, b2, s2)
    scatter(pidx, 0, b3, s3)
    wait_set(colA, rowA, wA, isemA)
    gather(colA, 0, b0, g0)
    gather(colA, 1, b1, g1)

    def quad(qq, cX, rX, wX, isemX, cY, rY, wY, isemY):
        gwait(b0, g0)
        scale(wX, 0, b0)
        swait(b2, s2)
        gather(cX, 2, b2, g2)
        scatter(rX, 0, b0, s0)
        gwait(b1, g1)
        scale(wX, 1, b1)
        swait(b3, s3)
        gather(cX, 3, b3, g3)
        scatter(rX, 1, b1, s1)
        gwait(b2, g2)
        scale(wX, 2, b2)
        scatter(rX, 2, b2, s2)
        gwait(b3, g3)
        scale(wX, 3, b3)
        scatter(rX, 3, b3, s3)
        wait_set(cY, rY, wY, isemY)
        swait(b0, s0)
        gather(cY, 0, b0, g0)
        swait(b1, s1)
        gather(cY, 1, b1, g1)
        load_set(lax.rem(qq + 2, NQUAD), cX, rX, wX, isemX)

    def super_body(p, _):
        quad(2 * p, colA, rowA, wA, isemA, colB, rowB, wB, isemB)
        quad(2 * p + 1, colB, rowB, wB, isemB, colA, rowA, wA, isemA)
        return 0

    lax.fori_loop(0, NQUAD // 2, super_body, 0)
    gwait(b0, g0)
    gwait(b1, g1)
    swait(b2, s2)
    swait(b3, s3)
    wait_set(colB, rowB, wB, isemB)

    plsc.subcore_barrier()

    @pl.when(cid == 0)
    def _():
        pltpu.sync_copy(acc.at[pl.ds(sid * RPT, RPT)],
                        out0.at[pl.ds(sid * RPT, RPT)])

    @pl.when(cid == 1)
    def _():
        pltpu.sync_copy(acc.at[pl.ds(sid * RPT, RPT)],
                        out1.at[pl.ds(sid * RPT, RPT)])


def _spmm_sc(support, row4, col4, w4):
    mesh = plsc.VectorSubcoreMesh(core_axis_name="c", subcore_axis_name="s")
    return pl.kernel(
        _spmm_kernel,
        out_type=(jax.ShapeDtypeStruct((NPAD, D), jnp.float32),
                  jax.ShapeDtypeStruct((NPAD, D), jnp.float32)),
        mesh=mesh,
        scratch_types=[
            pltpu.VMEM_SHARED((NPAD, D), jnp.float32),
            pltpu.VMEM((4, K), jnp.int32),
            pltpu.VMEM((4, K), jnp.int32),
            pltpu.VMEM((4, K), jnp.float32),
            pltpu.VMEM((4, K), jnp.int32),
            pltpu.VMEM((4, K), jnp.int32),
            pltpu.VMEM((4, K), jnp.float32),
            pltpu.VMEM((K, D), jnp.float32),
            pltpu.VMEM((K, D), jnp.float32),
            pltpu.VMEM((K, D), jnp.float32),
            pltpu.VMEM((K, D), jnp.float32),
            pltpu.VMEM((1, K), jnp.int32),
            pltpu.SemaphoreType.DMA,
            pltpu.SemaphoreType.DMA,
            pltpu.SemaphoreType.DMA,
            pltpu.SemaphoreType.DMA,
            pltpu.SemaphoreType.DMA,
            pltpu.SemaphoreType.DMA,
            pltpu.SemaphoreType.DMA,
            pltpu.SemaphoreType.DMA,
            pltpu.SemaphoreType.DMA,
            pltpu.SemaphoreType.DMA,
        ],
    )(support, row4, col4, w4)


def _combine_body(p0_ref, p1_ref, b_ref, o_ref):
    o_ref[...] = p0_ref[...] + p1_ref[...] + b_ref[...]


def _combine(p0, p1, b2d):
    bm = 2000
    return pl.pallas_call(
        _combine_body,
        grid=(N // bm,),
        in_specs=[
            pl.BlockSpec((bm, D), lambda i: (i, 0)),
            pl.BlockSpec((bm, D), lambda i: (i, 0)),
            pl.BlockSpec((1, D), lambda i: (0, 0)),
        ],
        out_specs=pl.BlockSpec((bm, D), lambda i: (i, 0)),
        out_shape=jax.ShapeDtypeStruct((N, D), jnp.float32),
    )(p0, p1, b2d)


@jax.jit
def kernel(x, edge_index, edge_weight, W, b):
    support = _matmul(x, W)
    npadE = EPAD - E
    fill = jnp.arange(npadE, dtype=jnp.int32)
    row_p = jnp.concatenate([edge_index[0], N + fill % (NPAD - N)])
    col_p = jnp.concatenate([edge_index[1], fill % N])
    w_p = jnp.concatenate([edge_weight, jnp.zeros((npadE,), jnp.float32)])
    row4 = row_p.reshape(NW, NQUAD, 4, K)
    col4 = col_p.reshape(NW, NQUAD, 4, K)
    w4 = w_p.reshape(NW, NQUAD, 4, K)
    p0, p1 = _spmm_sc(support, row4, col4, w4)
    return _combine(p0, p1, b.reshape(1, D))


# speedup vs baseline: 2.8872x; 1.0207x over previous
import jax
import jax.numpy as jnp
from jax import lax
from jax.experimental import pallas as pl
from jax.experimental.pallas import tpu as pltpu
from jax.experimental.pallas import tpu_sc as plsc

N = 10000
E = 320000
D = 128

NC = 2
NS = 16
NW = NC * NS
K = 80
NQUAD = 32
EPW = NQUAD * 4 * K
EPAD = NW * EPW
NPAD = 10240
RPT = NPAD // NS


def _matmul_body(x_ref, w_ref, o_ref):
    o_ref[...] = jnp.dot(x_ref[...], w_ref[...],
                         preferred_element_type=jnp.float32)


def _matmul(x, W):
    bm = 5000
    return pl.pallas_call(
        _matmul_body,
        grid=(N // bm,),
        in_specs=[
            pl.BlockSpec((bm, D), lambda i: (i, 0)),
            pl.BlockSpec((D, D), lambda i: (0, 0)),
        ],
        out_specs=pl.BlockSpec((bm, D), lambda i: (i, 0)),
        out_shape=jax.ShapeDtypeStruct((N, D), jnp.float32),
    )(x, W)


def _spmm_kernel(support, row, col, w, out0, out1, acc,
                 colA, rowA, wA, colB, rowB, wB,
                 b0, b1, b2, b3, pidx,
                 g0, g1, g2, g3, s0, s1, s2, s3, isemA, isemB):
    cid = lax.axis_index("c")
    sid = lax.axis_index("s")
    wid = sid * NC + cid

    bufs = (b0, b1, b2, b3)
    gsems = (g0, g1, g2, g3)
    ssems = (s0, s1, s2, s3)

    zeros16 = jnp.zeros((16,), jnp.float32)

    def zrow_b(buf):
        def zrow(r, _):
            for j in range(D // 16):
                buf[r, pl.ds(16 * j, 16)] = zeros16
            return 0
        lax.fori_loop(0, K, zrow, 0)

    zrow_b(b0)
    for z in range(RPT // K):
        pltpu.sync_copy(b0, acc.at[pl.ds(sid * RPT + z * K, K)])

    pbase = jnp.broadcast_to(N + sid * 10, (16,)).astype(jnp.int32)
    for t in range(K // 16):
        pidx[0, pl.ds(16 * t, 16)] = (
            lax.iota(jnp.int32, 16) + pbase + 16 * t)

    def load_set(qq, c, r, ww, sem):
        pltpu.async_copy(col.at[wid, qq], c, sem)
        pltpu.async_copy(row.at[wid, qq], r, sem)
        pltpu.async_copy(w.at[wid, qq], ww, sem)

    def wait_set(c, r, ww, sem):
        pltpu.make_async_copy(col, c, sem).wait()
        pltpu.make_async_copy(row, r, sem).wait()
        pltpu.make_async_copy(w, ww, sem).wait()

    def gather(cset, j, buf, gsem):
        pltpu.async_copy(support.at[cset.at[j]], buf, gsem)

    def gwait(buf, gsem):
        pltpu.make_async_copy(support, buf, gsem).wait()

    def scatter(rset, j, buf, ssem):
        pltpu.async_copy(buf, acc.at[rset.at[j]], ssem, add=True)

    def swait(buf, ssem):
        pltpu.make_async_copy(support, buf, ssem).wait()

    def scale(wset, j, buf):
        def body(g, _):
            wg = wset[j, pl.ds(16 * g, 16)]
            for e16 in range(16):
                wb = jnp.broadcast_to(wg[e16], (16,))
                e = 16 * g + e16
                for jj in range(D // 16):
                    buf[e, pl.ds(16 * jj, 16)] = (
                        buf[e, pl.ds(16 * jj, 16)] * wb)
            return 0
        lax.fori_loop(0, K // 16, body, 0)

    load_set(0, colA, rowA, wA, isemA)
    load_set(1, colB, rowB, wB, isemB)
    zrow_b(b2)
    zrow_b(b3)
    plsc.subcore_barrier()
    scatter(pidx, 0, b2, s2)
    scatter(pidx, 0, b3, s3)
    wait_set(colA, rowA, wA, isemA)
    gather(colA, 0, b0, g0)
    gather(colA, 1, b1, g1)

    def quad(qq, cX, rX, wX, isemX, cY, rY, wY, isemY):
        gwait(b0, g0)
        scale(wX, 0, b0)
        swait(b2, s2)
        gather(cX, 2, b2, g2)
        scatter(rX, 0, b0, s0)
        gwait(b1, g1)
        scale(wX, 1, b1)
        swait(b3, s3)
        gather(cX, 3, b3, g3)
        scatter(rX, 1, b1, s1)
        gwait(b2, g2)
        scale(wX, 2, b2)
        scatter(rX, 2, b2, s2)
        gwait(b3, g3)
        scale(wX, 3, b3)
        scatter(rX, 3, b3, s3)
        wait_set(cY, rY, wY, isemY)
        swait(b0, s0)
        gather(cY, 0, b0, g0)
        swait(b1, s1)
        gather(cY, 1, b1, g1)
        load_set(lax.rem(qq + 2, NQUAD), cX, rX, wX, isemX)

    def super_body(p, _):
        quad(2 * p, colA, rowA, wA, isemA, colB, rowB, wB, isemB)
        quad(2 * p + 1, colB, rowB, wB, isemB, colA, rowA, wA, isemA)
        return 0

    lax.fori_loop(0, NQUAD // 2, super_body, 0)
    gwait(b0, g0)
    gwait(b1, g1)
    swait(b2, s2)
    swait(b3, s3)
    wait_set(colB, rowB, wB, isemB)

    plsc.subcore_barrier()

    @pl.when(cid == 0)
    def _():
        pltpu.sync_copy(acc.at[pl.ds(sid * RPT, RPT)],
                        out0.at[pl.ds(sid * RPT, RPT)])

    @pl.when(cid == 1)
    def _():
        pltpu.sync_copy(acc.at[pl.ds(sid * RPT, RPT)],
                        out1.at[pl.ds(sid * RPT, RPT)])


def _spmm_sc(support, row4, col4, w4):
    mesh = plsc.VectorSubcoreMesh(core_axis_name="c", subcore_axis_name="s")
    return pl.kernel(
        _spmm_kernel,
        out_type=(jax.ShapeDtypeStruct((NPAD, D), jnp.float32),
                  jax.ShapeDtypeStruct((NPAD, D), jnp.float32)),
        mesh=mesh,
        scratch_types=[
            pltpu.VMEM_SHARED((NPAD, D), jnp.float32),
            pltpu.VMEM((4, K), jnp.int32),
            pltpu.VMEM((4, K), jnp.int32),
            pltpu.VMEM((4, K), jnp.float32),
            pltpu.VMEM((4, K), jnp.int32),
            pltpu.VMEM((4, K), jnp.int32),
            pltpu.VMEM((4, K), jnp.float32),
            pltpu.VMEM((K, D), jnp.float32),
            pltpu.VMEM((K, D), jnp.float32),
            pltpu.VMEM((K, D), jnp.float32),
            pltpu.VMEM((K, D), jnp.float32),
            pltpu.VMEM((1, K), jnp.int32),
            pltpu.SemaphoreType.DMA,
            pltpu.SemaphoreType.DMA,
            pltpu.SemaphoreType.DMA,
            pltpu.SemaphoreType.DMA,
            pltpu.SemaphoreType.DMA,
            pltpu.SemaphoreType.DMA,
            pltpu.SemaphoreType.DMA,
            pltpu.SemaphoreType.DMA,
            pltpu.SemaphoreType.DMA,
            pltpu.SemaphoreType.DMA,
        ],
    )(support, row4, col4, w4)


def _combine_body(p0_ref, p1_ref, b_ref, o_ref):
    o_ref[...] = p0_ref[...] + p1_ref[...] + b_ref[...]


def _combine(p0, p1, b2d):
    bm = 5000
    return pl.pallas_call(
        _combine_body,
        grid=(N // bm,),
        in_specs=[
            pl.BlockSpec((bm, D), lambda i: (i, 0)),
            pl.BlockSpec((bm, D), lambda i: (i, 0)),
            pl.BlockSpec((1, D), lambda i: (0, 0)),
        ],
        out_specs=pl.BlockSpec((bm, D), lambda i: (i, 0)),
        out_shape=jax.ShapeDtypeStruct((N, D), jnp.float32),
    )(p0, p1, b2d)


@jax.jit
def kernel(x, edge_index, edge_weight, W, b):
    support = _matmul(x, W)
    npadE = EPAD - E
    fill = jnp.arange(npadE, dtype=jnp.int32)
    row_p = jnp.concatenate([edge_index[0], N + fill % (NPAD - N)])
    col_p = jnp.concatenate([edge_index[1], fill % N])
    w_p = jnp.concatenate([edge_weight, jnp.zeros((npadE,), jnp.float32)])
    row4 = row_p.reshape(NW, NQUAD, 4, K)
    col4 = col_p.reshape(NW, NQUAD, 4, K)
    w4 = w_p.reshape(NW, NQUAD, 4, K)
    p0, p1 = _spmm_sc(support, row4, col4, w4)
    return _combine(p0, p1, b.reshape(1, D))
